# Initial kernel scaffold; baseline (speedup 1.0000x reference)
import jax
import jax.numpy as jnp
from jax.experimental import pallas as pl


def kernel(x_query, vis_mark, train, e_p, e_k, e_a):
    raise NotImplementedError("write your pallas kernel here")


# speedup vs baseline: 1.4730x; 1.4730x over previous
import jax
import jax.numpy as jnp
from jax import lax
from jax.experimental import pallas as pl
from jax.experimental.pallas import tpu as pltpu

_LOSS_W = 0.5
_ORTH_MU = 0.1
_BB = 8


def _body(p_ref, x_ref, k_ref, a_ref, out_ref, ploss_ref, oloss_ref):
    l = pl.program_id(0)
    bb = pl.program_id(1)

    out_ref[...] = jnp.broadcast_to(p_ref[...], out_ref.shape)

    @pl.when(bb == 0)
    def _losses():
        @pl.when(l == 0)
        def _init():
            ploss_ref[0, 0] = 0.0
            oloss_ref[0, 0] = 0.0

        K = k_ref[0]
        A = a_ref[0]
        x = x_ref[:, pl.ds(l, 1), :][:, 0, :]
        Bq = x.shape[0]
        topk = K.shape[0]

        num = lax.dot_general(x, A * K, (((1,), (1,)), ((), ())),
                              preferred_element_type=jnp.float32)
        den2 = lax.dot_general(x * x, A * A, (((1,), (1,)), ((), ())),
                               preferred_element_type=jnp.float32)
        kk = jnp.sum(K * K, axis=1)
        cos = num * lax.rsqrt(den2 * kk[None, :])
        loss = jnp.float32(Bq * topk) - jnp.sum(cos)

        def _ortho(t):
            nt = t * lax.rsqrt(jnp.sum(t * t, axis=1))[:, None]
            g = lax.dot_general(nt, nt, (((1,), (1,)), ((), ())),
                                preferred_element_type=jnp.float32)
            eye = jnp.eye(topk, dtype=jnp.float32)
            return jnp.sum((g - eye) ** 2) / (topk * topk) * 1e-06

        ploss_ref[0, 0] += _LOSS_W * loss
        oloss_ref[0, 0] += _ORTH_MU * (_ortho(K) + _ortho(A))


def kernel(x_query, vis_mark, train, e_p, e_k, e_a):
    L, T, topk, plen, D = e_p.shape
    Bq = x_query.shape[0]
    p0 = e_p[:, 0].reshape(L, topk * plen, D)
    k0 = e_k[:, 0]
    a0 = e_a[:, 0]

    grid = (L, Bq // _BB)
    out_shapes = (
        jax.ShapeDtypeStruct((L, Bq, topk * plen, D), jnp.float32),
        jax.ShapeDtypeStruct((1, 1), jnp.float32),
        jax.ShapeDtypeStruct((1, 1), jnp.float32),
    )
    in_specs = [
        pl.BlockSpec((1, topk * plen, D), lambda l, b: (l, 0, 0)),
        pl.BlockSpec((Bq, L, D), lambda l, b: (0, 0, 0)),
        pl.BlockSpec((1, topk, D), lambda l, b: (l, 0, 0)),
        pl.BlockSpec((1, topk, D), lambda l, b: (l, 0, 0)),
    ]
    out_specs = (
        pl.BlockSpec((1, _BB, topk * plen, D), lambda l, b: (l, b, 0, 0)),
        pl.BlockSpec(memory_space=pltpu.SMEM),
        pl.BlockSpec(memory_space=pltpu.SMEM),
    )
    out, ploss, oloss = pl.pallas_call(
        _body,
        grid=grid,
        in_specs=in_specs,
        out_specs=out_specs,
        out_shape=out_shapes,
    )(p0, x_query, k0, a0)
    return out, ploss[0, 0], oloss[0, 0]


# speedup vs baseline: 1.8033x; 1.2242x over previous
import jax
import jax.numpy as jnp
from jax import lax
from jax.experimental import pallas as pl
from jax.experimental.pallas import tpu as pltpu

_LOSS_W = 0.5
_ORTH_MU = 0.1
_BB = 16


def _body(p_ref, x_ref, k_ref, a_ref, out_ref, ploss_ref, oloss_ref):
    l = pl.program_id(0)
    bb = pl.program_id(1)

    out_ref[...] = jnp.broadcast_to(p_ref[...], out_ref.shape)

    @pl.when(bb == 0)
    def _losses():
        @pl.when(l == 0)
        def _init():
            ploss_ref[0, 0] = 0.0
            oloss_ref[0, 0] = 0.0

        K = k_ref[0]
        A = a_ref[0]
        x = x_ref[:, pl.ds(l, 1), :][:, 0, :]
        Bq = x.shape[0]
        topk = K.shape[0]

        num = lax.dot_general(x, A * K, (((1,), (1,)), ((), ())),
                              preferred_element_type=jnp.float32)
        den2 = lax.dot_general(x * x, A * A, (((1,), (1,)), ((), ())),
                               preferred_element_type=jnp.float32)
        kk = jnp.sum(K * K, axis=1)
        cos = num * lax.rsqrt(den2 * kk[None, :])
        loss = jnp.float32(Bq * topk) - jnp.sum(cos)

        def _ortho(t):
            nt = t * lax.rsqrt(jnp.sum(t * t, axis=1))[:, None]
            g = lax.dot_general(nt, nt, (((1,), (1,)), ((), ())),
                                preferred_element_type=jnp.float32)
            eye = jnp.eye(topk, dtype=jnp.float32)
            return jnp.sum((g - eye) ** 2) / (topk * topk) * 1e-06

        ploss_ref[0, 0] += _LOSS_W * loss
        oloss_ref[0, 0] += _ORTH_MU * (_ortho(K) + _ortho(A))


def kernel(x_query, vis_mark, train, e_p, e_k, e_a):
    L, T, topk, plen, D = e_p.shape
    Bq = x_query.shape[0]
    p0 = e_p[:, 0].reshape(L, topk * plen, D)
    k0 = e_k[:, 0]
    a0 = e_a[:, 0]

    grid = (L, Bq // _BB)
    out_shapes = (
        jax.ShapeDtypeStruct((L, Bq, topk * plen, D), jnp.float32),
        jax.ShapeDtypeStruct((1, 1), jnp.float32),
        jax.ShapeDtypeStruct((1, 1), jnp.float32),
    )
    in_specs = [
        pl.BlockSpec((1, topk * plen, D), lambda l, b: (l, 0, 0)),
        pl.BlockSpec((Bq, L, D), lambda l, b: (0, 0, 0)),
        pl.BlockSpec((1, topk, D), lambda l, b: (l, 0, 0)),
        pl.BlockSpec((1, topk, D), lambda l, b: (l, 0, 0)),
    ]
    out_specs = (
        pl.BlockSpec((1, _BB, topk * plen, D), lambda l, b: (l, b, 0, 0)),
        pl.BlockSpec(memory_space=pltpu.SMEM),
        pl.BlockSpec(memory_space=pltpu.SMEM),
    )
    out, ploss, oloss = pl.pallas_call(
        _body,
        grid=grid,
        in_specs=in_specs,
        out_specs=out_specs,
        out_shape=out_shapes,
    )(p0, x_query, k0, a0)
    return out, ploss[0, 0], oloss[0, 0]


# speedup vs baseline: 2.3807x; 1.3202x over previous
import jax
import jax.numpy as jnp
from jax import lax
from jax.experimental import pallas as pl
from jax.experimental.pallas import tpu as pltpu

_LOSS_W = 0.5
_ORTH_MU = 0.1
_BB = 32


def _body(p_ref, x_ref, k_ref, a_ref, out_ref, ploss_ref, oloss_ref):
    l = pl.program_id(0)
    bb = pl.program_id(1)

    out_ref[...] = jnp.broadcast_to(p_ref[...], out_ref.shape)

    @pl.when(bb == 0)
    def _losses():
        @pl.when(l == 0)
        def _init():
            ploss_ref[0, 0] = 0.0
            oloss_ref[0, 0] = 0.0

        K = k_ref[0]
        A = a_ref[0]
        x = x_ref[:, pl.ds(l, 1), :][:, 0, :]
        Bq = x.shape[0]
        topk = K.shape[0]

        num = lax.dot_general(x, A * K, (((1,), (1,)), ((), ())),
                              preferred_element_type=jnp.float32)
        den2 = lax.dot_general(x * x, A * A, (((1,), (1,)), ((), ())),
                               preferred_element_type=jnp.float32)
        kk = jnp.sum(K * K, axis=1)
        cos = num * lax.rsqrt(den2 * kk[None, :])
        loss = jnp.float32(Bq * topk) - jnp.sum(cos)

        def _ortho(t):
            nt = t * lax.rsqrt(jnp.sum(t * t, axis=1))[:, None]
            g = lax.dot_general(nt, nt, (((1,), (1,)), ((), ())),
                                preferred_element_type=jnp.float32)
            eye = jnp.eye(topk, dtype=jnp.float32)
            return jnp.sum((g - eye) ** 2) / (topk * topk) * 1e-06

        ploss_ref[0, 0] += _LOSS_W * loss
        oloss_ref[0, 0] += _ORTH_MU * (_ortho(K) + _ortho(A))


def kernel(x_query, vis_mark, train, e_p, e_k, e_a):
    L, T, topk, plen, D = e_p.shape
    Bq = x_query.shape[0]
    p0 = e_p[:, 0].reshape(L, topk * plen, D)
    k0 = e_k[:, 0]
    a0 = e_a[:, 0]

    grid = (L, Bq // _BB)
    out_shapes = (
        jax.ShapeDtypeStruct((L, Bq, topk * plen, D), jnp.float32),
        jax.ShapeDtypeStruct((1, 1), jnp.float32),
        jax.ShapeDtypeStruct((1, 1), jnp.float32),
    )
    in_specs = [
        pl.BlockSpec((1, topk * plen, D), lambda l, b: (l, 0, 0)),
        pl.BlockSpec((Bq, L, D), lambda l, b: (0, 0, 0)),
        pl.BlockSpec((1, topk, D), lambda l, b: (l, 0, 0)),
        pl.BlockSpec((1, topk, D), lambda l, b: (l, 0, 0)),
    ]
    out_specs = (
        pl.BlockSpec((1, _BB, topk * plen, D), lambda l, b: (l, b, 0, 0)),
        pl.BlockSpec(memory_space=pltpu.SMEM),
        pl.BlockSpec(memory_space=pltpu.SMEM),
    )
    out, ploss, oloss = pl.pallas_call(
        _body,
        grid=grid,
        in_specs=in_specs,
        out_specs=out_specs,
        out_shape=out_shapes,
    )(p0, x_query, k0, a0)
    return out, ploss[0, 0], oloss[0, 0]


# speedup vs baseline: 2.6190x; 1.1001x over previous
import jax
import jax.numpy as jnp
from jax import lax
from jax.experimental import pallas as pl
from jax.experimental.pallas import tpu as pltpu

_LOSS_W = 0.5
_ORTH_MU = 0.1
_BB = 64


def _body(p_ref, x_ref, k_ref, a_ref, out_ref, ploss_ref, oloss_ref):
    l = pl.program_id(0)
    bb = pl.program_id(1)

    out_ref[...] = jnp.broadcast_to(p_ref[...], out_ref.shape)

    @pl.when(bb == 0)
    def _losses():
        @pl.when(l == 0)
        def _init():
            ploss_ref[0, 0] = 0.0
            oloss_ref[0, 0] = 0.0

        K = k_ref[0]
        A = a_ref[0]
        x = x_ref[:, pl.ds(l, 1), :][:, 0, :]
        Bq = x.shape[0]
        topk = K.shape[0]

        num = lax.dot_general(x, A * K, (((1,), (1,)), ((), ())),
                              preferred_element_type=jnp.float32)
        den2 = lax.dot_general(x * x, A * A, (((1,), (1,)), ((), ())),
                               preferred_element_type=jnp.float32)
        kk = jnp.sum(K * K, axis=1)
        cos = num * lax.rsqrt(den2 * kk[None, :])
        loss = jnp.float32(Bq * topk) - jnp.sum(cos)

        def _ortho(t):
            nt = t * lax.rsqrt(jnp.sum(t * t, axis=1))[:, None]
            g = lax.dot_general(nt, nt, (((1,), (1,)), ((), ())),
                                preferred_element_type=jnp.float32)
            eye = jnp.eye(topk, dtype=jnp.float32)
            return jnp.sum((g - eye) ** 2) / (topk * topk) * 1e-06

        ploss_ref[0, 0] += _LOSS_W * loss
        oloss_ref[0, 0] += _ORTH_MU * (_ortho(K) + _ortho(A))


def kernel(x_query, vis_mark, train, e_p, e_k, e_a):
    L, T, topk, plen, D = e_p.shape
    Bq = x_query.shape[0]
    p0 = e_p[:, 0].reshape(L, topk * plen, D)
    k0 = e_k[:, 0]
    a0 = e_a[:, 0]

    grid = (L, Bq // _BB)
    out_shapes = (
        jax.ShapeDtypeStruct((L, Bq, topk * plen, D), jnp.float32),
        jax.ShapeDtypeStruct((1, 1), jnp.float32),
        jax.ShapeDtypeStruct((1, 1), jnp.float32),
    )
    in_specs = [
        pl.BlockSpec((1, topk * plen, D), lambda l, b: (l, 0, 0)),
        pl.BlockSpec((Bq, L, D), lambda l, b: (0, 0, 0)),
        pl.BlockSpec((1, topk, D), lambda l, b: (l, 0, 0)),
        pl.BlockSpec((1, topk, D), lambda l, b: (l, 0, 0)),
    ]
    out_specs = (
        pl.BlockSpec((1, _BB, topk * plen, D), lambda l, b: (l, b, 0, 0)),
        pl.BlockSpec(memory_space=pltpu.SMEM),
        pl.BlockSpec(memory_space=pltpu.SMEM),
    )
    out, ploss, oloss = pl.pallas_call(
        _body,
        grid=grid,
        in_specs=in_specs,
        out_specs=out_specs,
        out_shape=out_shapes,
    )(p0, x_query, k0, a0)
    return out, ploss[0, 0], oloss[0, 0]
